# baseline (speedup 1.0000x reference)
import jax
import jax.numpy as jnp
from jax.experimental import pallas as pl
from jax.experimental.pallas import tpu as pltpu


_FC_FFN = 256
_FC_MOE = 1536
_PREC = jax.lax.Precision.DEFAULT


def _ffn_body(x_ref, w1_ref, b1_ref, w2_ref, b2_ref, o_ref):
    f = pl.program_id(0)

    @pl.when(f == 0)
    def _():
        o_ref[:] = jnp.zeros_like(o_ref) + b2_ref[:]

    h = jnp.dot(x_ref[:], w1_ref[:], precision=_PREC,
                preferred_element_type=jnp.float32)
    h = jax.nn.gelu((h + b1_ref[:]).astype(jnp.bfloat16))
    o_ref[:] += jnp.dot(h, w2_ref[:], precision=_PREC,
                        preferred_element_type=jnp.float32)


def _moe_body(x_ref, wg_ref, ffnb_ref, we1_ref, be1_ref, we2_ref, be2_ref,
              o_ref, gall_ref):
    e = pl.program_id(0)
    f = pl.program_id(1)
    E = wg_ref.shape[1]

    @pl.when(jnp.logical_and(e == 0, f == 0))
    def _():
        g = jax.nn.softmax(wg_ref[:], axis=-1)
        lane = jax.lax.broadcasted_iota(jnp.int32, g.shape, 1)
        for i in range(E):
            gall_ref[i] = jnp.sum(jnp.where(lane == i, g, 0.0), axis=1,
                                  keepdims=True)
        o_ref[:] = ffnb_ref[:] + jnp.dot(
            g, be2_ref[:], precision=_PREC,
            preferred_element_type=jnp.float32)

    ge = gall_ref[e]
    h = jnp.dot(x_ref[:], we1_ref[0], precision=_PREC,
                preferred_element_type=jnp.float32)
    h = jax.nn.gelu((h + be1_ref[0]).astype(jnp.bfloat16))
    o_ref[:] += ge * jnp.dot(h, we2_ref[0], precision=_PREC,
                             preferred_element_type=jnp.float32)


def _shared_ffn(x2, W1, b1, W2, b2):
    T, D = x2.shape
    F = W1.shape[1]
    fc = _FC_FFN
    return pl.pallas_call(
        _ffn_body,
        grid=(F // fc,),
        in_specs=[
            pl.BlockSpec((T, D), lambda f: (0, 0)),
            pl.BlockSpec((D, fc), lambda f: (0, f)),
            pl.BlockSpec((1, fc), lambda f: (0, f)),
            pl.BlockSpec((fc, D), lambda f: (f, 0)),
            pl.BlockSpec((1, D), lambda f: (0, 0)),
        ],
        out_specs=pl.BlockSpec((T, D), lambda f: (0, 0)),
        out_shape=jax.ShapeDtypeStruct((T, D), jnp.float32),
        compiler_params=pltpu.CompilerParams(
            dimension_semantics=("arbitrary",)),
    )(x2, W1, b1.reshape(1, F), W2, b2.reshape(1, D))


def _moe_combine(x2, wg2, ffn_out, We1, be1, We2, be2):
    T, D = x2.shape
    Tb, E = wg2.shape
    F = We1.shape[2]
    fc = _FC_MOE
    return pl.pallas_call(
        _moe_body,
        grid=(E, F // fc),
        in_specs=[
            pl.BlockSpec((Tb, D), lambda e, f: (1, 0)),
            pl.BlockSpec((Tb, E), lambda e, f: (0, 0)),
            pl.BlockSpec((Tb, D), lambda e, f: (1, 0)),
            pl.BlockSpec((1, D, fc), lambda e, f: (e, 0, f)),
            pl.BlockSpec((1, 1, fc), lambda e, f: (e, 0, f)),
            pl.BlockSpec((1, fc, D), lambda e, f: (e, f, 0)),
            pl.BlockSpec((E, D), lambda e, f: (0, 0)),
        ],
        out_specs=pl.BlockSpec((Tb, D), lambda e, f: (1, 0)),
        out_shape=jax.ShapeDtypeStruct((T, D), jnp.float32),
        input_output_aliases={2: 0},
        scratch_shapes=[
            pltpu.VMEM((E, Tb, 1), jnp.float32),
        ],
        compiler_params=pltpu.CompilerParams(
            dimension_semantics=("arbitrary", "arbitrary")),
    )(x2, wg2, ffn_out, We1, be1.reshape(E, 1, F), We2, be2)


def kernel(x, id, weight, W1, b1, W2, b2, We1, be1, We2, be2):
    B, T, D = x.shape
    x2 = x.reshape(T, D)
    wg2 = weight.reshape(weight.shape[1], weight.shape[2])

    ffn_out = _shared_ffn(x2, W1, b1, W2, b2)
    out = _moe_combine(x2, wg2, ffn_out, We1, be1, We2, be2)
    return out.reshape(B, T, D)


# speedup vs baseline: 1.1145x; 1.1145x over previous
import jax
import jax.numpy as jnp
from jax.experimental import pallas as pl
from jax.experimental.pallas import tpu as pltpu


_FC_FFN = 512
_FC_MOE = 1536
_PREC = jax.lax.Precision.DEFAULT


def _ffn_body(x_ref, w1_ref, b1_ref, w2_ref, b2_ref, o_ref):
    f = pl.program_id(0)

    @pl.when(f == 0)
    def _():
        o_ref[:] = jnp.zeros_like(o_ref) + b2_ref[:]

    h = jnp.dot(x_ref[:], w1_ref[:], precision=_PREC,
                preferred_element_type=jnp.float32)
    h = jax.nn.gelu((h + b1_ref[:]).astype(jnp.bfloat16))
    o_ref[:] += jnp.dot(h, w2_ref[:], precision=_PREC,
                        preferred_element_type=jnp.float32)


def _moe_body(x_ref, wg_ref, ffnb_ref, we1_ref, be1_ref, we2_ref, be2_ref,
              o_ref, gall_ref):
    e = pl.program_id(0)
    f = pl.program_id(1)
    E = wg_ref.shape[1]

    @pl.when(jnp.logical_and(e == 0, f == 0))
    def _():
        g = jax.nn.softmax(wg_ref[:], axis=-1)
        lane = jax.lax.broadcasted_iota(jnp.int32, g.shape, 1)
        for i in range(E):
            gall_ref[i] = jnp.sum(jnp.where(lane == i, g, 0.0), axis=1,
                                  keepdims=True)
        o_ref[:] = ffnb_ref[:] + jnp.dot(
            g, be2_ref[:], precision=_PREC,
            preferred_element_type=jnp.float32)

    ge = gall_ref[e]
    h = jnp.dot(x_ref[:], we1_ref[0], precision=_PREC,
                preferred_element_type=jnp.float32)
    h = jax.nn.gelu((h + be1_ref[0]).astype(jnp.bfloat16))
    o_ref[:] += ge * jnp.dot(h, we2_ref[0], precision=_PREC,
                             preferred_element_type=jnp.float32)


def _shared_ffn(x2, W1, b1, W2, b2):
    T, D = x2.shape
    F = W1.shape[1]
    fc = _FC_FFN
    return pl.pallas_call(
        _ffn_body,
        grid=(F // fc,),
        in_specs=[
            pl.BlockSpec((T, D), lambda f: (0, 0)),
            pl.BlockSpec((D, fc), lambda f: (0, f)),
            pl.BlockSpec((1, fc), lambda f: (0, f)),
            pl.BlockSpec((fc, D), lambda f: (f, 0)),
            pl.BlockSpec((1, D), lambda f: (0, 0)),
        ],
        out_specs=pl.BlockSpec((T, D), lambda f: (0, 0)),
        out_shape=jax.ShapeDtypeStruct((T, D), jnp.float32),
        compiler_params=pltpu.CompilerParams(
            dimension_semantics=("arbitrary",)),
    )(x2, W1, b1.reshape(1, F), W2, b2.reshape(1, D))


def _moe_combine(x2, wg2, ffn_out, We1, be1, We2, be2):
    T, D = x2.shape
    Tb, E = wg2.shape
    F = We1.shape[2]
    fc = _FC_MOE
    return pl.pallas_call(
        _moe_body,
        grid=(E, F // fc),
        in_specs=[
            pl.BlockSpec((Tb, D), lambda e, f: (1, 0)),
            pl.BlockSpec((Tb, E), lambda e, f: (0, 0)),
            pl.BlockSpec((Tb, D), lambda e, f: (1, 0)),
            pl.BlockSpec((1, D, fc), lambda e, f: (e, 0, f)),
            pl.BlockSpec((1, 1, fc), lambda e, f: (e, 0, f)),
            pl.BlockSpec((1, fc, D), lambda e, f: (e, f, 0)),
            pl.BlockSpec((E, D), lambda e, f: (0, 0)),
        ],
        out_specs=pl.BlockSpec((Tb, D), lambda e, f: (1, 0)),
        out_shape=jax.ShapeDtypeStruct((T, D), jnp.float32),
        input_output_aliases={2: 0},
        scratch_shapes=[
            pltpu.VMEM((E, Tb, 1), jnp.float32),
        ],
        compiler_params=pltpu.CompilerParams(
            dimension_semantics=("arbitrary", "arbitrary")),
    )(x2, wg2, ffn_out, We1, be1.reshape(E, 1, F), We2, be2)


def kernel(x, id, weight, W1, b1, W2, b2, We1, be1, We2, be2):
    B, T, D = x.shape
    x2 = x.reshape(T, D)
    wg2 = weight.reshape(weight.shape[1], weight.shape[2])

    ffn_out = _shared_ffn(x2, W1, b1, W2, b2)
    out = _moe_combine(x2, wg2, ffn_out, We1, be1, We2, be2)
    return out.reshape(B, T, D)
